# Initial kernel scaffold; baseline (speedup 1.0000x reference)
import jax
import jax.numpy as jnp
from jax.experimental import pallas as pl


def kernel(pos_u, pos_v, neg_v, table):
    raise NotImplementedError("write your pallas kernel here")


# speedup vs baseline: 1.3652x; 1.3652x over previous
import jax
import jax.numpy as jnp
from jax import lax
from jax.experimental import pallas as pl
from jax.experimental.pallas import tpu as pltpu
from jax.experimental.pallas import tpu_sc as plsc

VOCAB = 1000000
DIM = 64
P = 4096
L = 50
LU = 200
NW = 32
GPW = P // NW
CHUNK = 100
CPW = GPW // 2
LANES = 16
NCOL = DIM // LANES


def _row_sum(buf_ref, r0, n):
    z = jnp.zeros((LANES,), jnp.float32)

    def body(j, carry):
        r = r0 + j
        return tuple(carry[k] + buf_ref[r, pl.ds(k * LANES, LANES)]
                     for k in range(NCOL))

    return lax.fori_loop(0, n, body, (z,) * NCOL)


def _seg_body(posu_hbm, posv_hbm, negv_hbm, table_hbm,
              negsum_hbm, pospart_hbm, posusum_hbm,
              idx_ref, idxu_ref, buf_ref, acc_ref, vec_ref, uvec_ref):
    core = lax.axis_index("core")
    sub = lax.axis_index("subcore")
    wid = sub * 2 + core

    pltpu.sync_copy(negv_hbm.at[wid], idx_ref)

    @pl.loop(0, CPW)
    def _neg(c):
        pltpu.sync_copy(table_hbm.at[idx_ref.at[c]], buf_ref)
        for gg in range(2):
            s = _row_sum(buf_ref, gg * L, L)
            g = c * 2 + gg
            for k in range(NCOL):
                acc_ref[g, pl.ds(k * LANES, LANES)] = s[k]

    pltpu.sync_copy(acc_ref, negsum_hbm.at[pl.ds(wid * GPW, GPW)])

    pltpu.sync_copy(posv_hbm.at[wid], idx_ref)
    for k in range(NCOL):
        vec_ref[0, pl.ds(k * LANES, LANES)] = jnp.zeros((LANES,), jnp.float32)

    @pl.loop(0, CPW)
    def _pos(c):
        pltpu.sync_copy(table_hbm.at[idx_ref.at[c]], buf_ref)
        s = _row_sum(buf_ref, 0, CHUNK)
        for k in range(NCOL):
            vec_ref[0, pl.ds(k * LANES, LANES)] += s[k]

    pltpu.sync_copy(vec_ref, pospart_hbm.at[pl.ds(wid, 1)])

    @pl.when(wid == 0)
    def _posu():
        pltpu.sync_copy(posu_hbm, idxu_ref)
        for k in range(NCOL):
            uvec_ref[0, pl.ds(k * LANES, LANES)] = jnp.zeros((LANES,), jnp.float32)
        for c in range(LU // CHUNK):
            pltpu.sync_copy(table_hbm.at[idxu_ref.at[c]], buf_ref)
            s = _row_sum(buf_ref, 0, CHUNK)
            for k in range(NCOL):
                uvec_ref[0, pl.ds(k * LANES, LANES)] += s[k]
        pltpu.sync_copy(uvec_ref, posusum_hbm)


def _finish_body(negsum_ref, pospart_ref, posusum_ref, out_ref):
    node = posusum_ref[...] / float(LU)
    pos_total = jnp.sum(pospart_ref[...], axis=0, keepdims=True)
    inv = 1.0 / (float(L) * float(P))
    pos_loss = jnp.sum(node * pos_total) * inv
    nr = jnp.sum(negsum_ref[...] * node, axis=1) * inv
    val = jnp.exp(nr) * (nr > 0.0).astype(jnp.float32)
    s = jnp.sum(val)
    out_ref[...] = jnp.broadcast_to(jnp.log(1.0 + s) - pos_loss, (1, 1))


def _seg_call(posu2, posv3, negv3, table):
    mesh = plsc.VectorSubcoreMesh(core_axis_name="core", subcore_axis_name="subcore")
    f = pl.kernel(
        _seg_body,
        compiler_params=pltpu.CompilerParams(use_tc_tiling_on_sc=False),
        out_type=[
            jax.ShapeDtypeStruct((P, DIM), jnp.float32),
            jax.ShapeDtypeStruct((NW, DIM), jnp.float32),
            jax.ShapeDtypeStruct((1, DIM), jnp.float32),
        ],
        mesh=mesh,
        scratch_types=[
            pltpu.VMEM((CPW, CHUNK), jnp.int32),
            pltpu.VMEM((LU // CHUNK, CHUNK), jnp.int32),
            pltpu.VMEM((CHUNK, DIM), jnp.float32),
            pltpu.VMEM((GPW, DIM), jnp.float32),
            pltpu.VMEM((1, DIM), jnp.float32),
            pltpu.VMEM((1, DIM), jnp.float32),
        ],
    )
    return f(posu2, posv3, negv3, table)


def _finish_call(negsum, pospart, posusum):
    return pl.pallas_call(
        _finish_body,
        out_shape=jax.ShapeDtypeStruct((1, 1), jnp.float32),
    )(negsum, pospart, posusum)


def kernel(pos_u, pos_v, neg_v, table):
    posu2 = pos_u.astype(jnp.int32).reshape(LU // CHUNK, CHUNK)
    posv3 = pos_v.astype(jnp.int32).reshape(NW, CPW, CHUNK)
    negv3 = neg_v.astype(jnp.int32).reshape(NW, CPW, CHUNK)
    negsum, pospart, posusum = _seg_call(posu2, posv3, negv3, table)
    out = _finish_call(negsum, pospart, posusum)
    return out[0, 0]


# speedup vs baseline: 1.5090x; 1.1053x over previous
import jax
import jax.numpy as jnp
from jax import lax
from jax.experimental import pallas as pl
from jax.experimental.pallas import tpu as pltpu
from jax.experimental.pallas import tpu_sc as plsc

VOCAB = 1000000
DIM = 64
P = 4096
L = 50
LU = 200
NW = 32
GPW = P // NW
CHUNK = 100
CPW = GPW // 2
LANES = 16
NCOL = DIM // LANES


def _row_sum(buf_ref, r0, n):
    z = jnp.zeros((LANES,), jnp.float32)

    def body(j, carry):
        r = r0 + j
        return tuple(carry[k] + buf_ref[r, pl.ds(k * LANES, LANES)]
                     for k in range(NCOL))

    return lax.fori_loop(0, n, body, (z,) * NCOL)


def _gather_pass(table_hbm, idx_ref, buf0, buf1, sem0, sem1, nchunks, process):
    bufs = (buf0, buf1)
    sems = (sem0, sem1)

    def start(c, b):
        pltpu.async_copy(table_hbm.at[idx_ref.at[c]], bufs[b], sems[b])

    def wait(c, b):
        pltpu.make_async_copy(table_hbm.at[idx_ref.at[c]], bufs[b], sems[b]).wait()

    start(0, 0)
    start(1, 1)

    @pl.loop(0, nchunks - 2, step=2)
    def _(c):
        wait(c, 0)
        process(buf0, c)
        start(c + 2, 0)
        wait(c + 1, 1)
        process(buf1, c + 1)
        start(c + 3, 1)

    wait(nchunks - 2, 0)
    process(buf0, nchunks - 2)
    wait(nchunks - 1, 1)
    process(buf1, nchunks - 1)


def _seg_body(posu_hbm, posv_hbm, negv_hbm, table_hbm,
              negsum_hbm, pospart_hbm, posusum_hbm,
              idxn_ref, idxp_ref, idxu_ref, buf0, buf1,
              acc_ref, vec_ref, uvec_ref, sem0, sem1, semi):
    core = lax.axis_index("core")
    sub = lax.axis_index("subcore")
    wid = sub * 2 + core

    cpi_n = pltpu.async_copy(negv_hbm.at[wid], idxn_ref, semi)
    cpi_p = pltpu.async_copy(posv_hbm.at[wid], idxp_ref, semi)
    cpi_u = pltpu.async_copy(posu_hbm, idxu_ref, semi)
    cpi_n.wait()
    cpi_p.wait()
    cpi_u.wait()

    def proc_neg(buf_ref, c):
        for gg in range(2):
            s = _row_sum(buf_ref, gg * L, L)
            g = c * 2 + gg
            for k in range(NCOL):
                acc_ref[g, pl.ds(k * LANES, LANES)] = s[k]

    _gather_pass(table_hbm, idxn_ref, buf0, buf1, sem0, sem1, CPW, proc_neg)
    pltpu.sync_copy(acc_ref, negsum_hbm.at[pl.ds(wid * GPW, GPW)])

    for k in range(NCOL):
        vec_ref[0, pl.ds(k * LANES, LANES)] = jnp.zeros((LANES,), jnp.float32)

    def proc_pos(buf_ref, c):
        s = _row_sum(buf_ref, 0, CHUNK)
        for k in range(NCOL):
            vec_ref[0, pl.ds(k * LANES, LANES)] += s[k]

    _gather_pass(table_hbm, idxp_ref, buf0, buf1, sem0, sem1, CPW, proc_pos)
    pltpu.sync_copy(vec_ref, pospart_hbm.at[pl.ds(wid, 1)])

    @pl.when(wid == 0)
    def _posu():
        for k in range(NCOL):
            uvec_ref[0, pl.ds(k * LANES, LANES)] = jnp.zeros((LANES,), jnp.float32)
        for c in range(LU // CHUNK):
            pltpu.sync_copy(table_hbm.at[idxu_ref.at[c]], buf0)
            s = _row_sum(buf0, 0, CHUNK)
            for k in range(NCOL):
                uvec_ref[0, pl.ds(k * LANES, LANES)] += s[k]
        pltpu.sync_copy(uvec_ref, posusum_hbm)


def _finish_body(negsum_ref, pospart_ref, posusum_ref, out_ref):
    node = posusum_ref[...] / float(LU)
    pos_total = jnp.sum(pospart_ref[...], axis=0, keepdims=True)
    inv = 1.0 / (float(L) * float(P))
    pos_loss = jnp.sum(node * pos_total) * inv
    nr = jnp.sum(negsum_ref[...] * node, axis=1) * inv
    val = jnp.exp(nr) * (nr > 0.0).astype(jnp.float32)
    s = jnp.sum(val)
    out_ref[...] = jnp.broadcast_to(jnp.log(1.0 + s) - pos_loss, (1, 1))


def _seg_call(posu2, posv3, negv3, table):
    mesh = plsc.VectorSubcoreMesh(core_axis_name="core", subcore_axis_name="subcore")
    f = pl.kernel(
        _seg_body,
        compiler_params=pltpu.CompilerParams(use_tc_tiling_on_sc=False),
        out_type=[
            jax.ShapeDtypeStruct((P, DIM), jnp.float32),
            jax.ShapeDtypeStruct((NW, DIM), jnp.float32),
            jax.ShapeDtypeStruct((1, DIM), jnp.float32),
        ],
        mesh=mesh,
        scratch_types=[
            pltpu.VMEM((CPW, CHUNK), jnp.int32),
            pltpu.VMEM((CPW, CHUNK), jnp.int32),
            pltpu.VMEM((LU // CHUNK, CHUNK), jnp.int32),
            pltpu.VMEM((CHUNK, DIM), jnp.float32),
            pltpu.VMEM((CHUNK, DIM), jnp.float32),
            pltpu.VMEM((GPW, DIM), jnp.float32),
            pltpu.VMEM((1, DIM), jnp.float32),
            pltpu.VMEM((1, DIM), jnp.float32),
            pltpu.SemaphoreType.DMA,
            pltpu.SemaphoreType.DMA,
            pltpu.SemaphoreType.DMA,
        ],
    )
    return f(posu2, posv3, negv3, table)


def _finish_call(negsum, pospart, posusum):
    return pl.pallas_call(
        _finish_body,
        out_shape=jax.ShapeDtypeStruct((1, 1), jnp.float32),
    )(negsum, pospart, posusum)


def kernel(pos_u, pos_v, neg_v, table):
    posu2 = pos_u.astype(jnp.int32).reshape(LU // CHUNK, CHUNK)
    posv3 = pos_v.astype(jnp.int32).reshape(NW, CPW, CHUNK)
    negv3 = neg_v.astype(jnp.int32).reshape(NW, CPW, CHUNK)
    negsum, pospart, posusum = _seg_call(posu2, posv3, negv3, table)
    out = _finish_call(negsum, pospart, posusum)
    return out[0, 0]


# speedup vs baseline: 3.2399x; 2.1471x over previous
import jax
import jax.numpy as jnp
from jax import lax
from jax.experimental import pallas as pl
from jax.experimental.pallas import tpu as pltpu
from jax.experimental.pallas import tpu_sc as plsc

VOCAB = 1000000
DIM = 64
P = 4096
L = 50
LU = 200
NW = 32
GPW = P // NW
OPW = GPW * L
LANES = 16
INV = 1.0 / (float(L) * float(P))
SWEEP_BLK = 4096


def _node_body(idx_ref, tbl_ref, out_ref):
    i = pl.program_id(0)

    @pl.when(i == 0)
    def _():
        out_ref[...] = jnp.zeros_like(out_ref)

    lane = idx_ref[i] % 128
    mask = jax.lax.broadcasted_iota(jnp.int32, (DIM, 128), 1) == lane
    col = jnp.sum(jnp.where(mask, tbl_ref[...], 0.0), axis=1, keepdims=True)
    out_ref[...] += col

    @pl.when(i == LU - 1)
    def _():
        out_ref[...] = out_ref[...] / float(LU)


def _node_call(pos_u, table_t):
    return pl.pallas_call(
        _node_body,
        grid_spec=pltpu.PrefetchScalarGridSpec(
            num_scalar_prefetch=1,
            grid=(LU,),
            in_specs=[
                pl.BlockSpec((DIM, 128), lambda i, idx_ref: (0, idx_ref[i] // 128)),
            ],
            out_specs=pl.BlockSpec((DIM, 1), lambda i, idx_ref: (0, 0)),
        ),
        out_shape=jax.ShapeDtypeStruct((DIM, 1), jnp.float32),
    )(pos_u, table_t)


def _sweep_body(tbl_ref, node_ref, s_ref):
    s_ref[...] = jnp.sum(tbl_ref[...] * node_ref[...], axis=0)


def _sweep_call(table_t, node):
    grid = (VOCAB + SWEEP_BLK - 1) // SWEEP_BLK
    return pl.pallas_call(
        _sweep_body,
        grid=(grid,),
        in_specs=[
            pl.BlockSpec((DIM, SWEEP_BLK), lambda i: (0, i)),
            pl.BlockSpec((DIM, 1), lambda i: (0, 0)),
        ],
        out_specs=pl.BlockSpec((SWEEP_BLK,), lambda i: (i,)),
        out_shape=jax.ShapeDtypeStruct((VOCAB,), jnp.float32),
    )(table_t, node)


GCH = 128
NCH = OPW // GCH


def _fetch_s(s_hbm, idx_ref, sbuf_ref, sem):
    @pl.loop(0, NCH)
    def _fire(c):
        pltpu.async_copy(
            s_hbm.at[idx_ref.at[pl.ds(c * GCH, GCH)]],
            sbuf_ref.at[pl.ds(c * GCH, GCH)], sem)

    @pl.loop(0, NCH)
    def _drain(c):
        pltpu.make_async_copy(
            s_hbm.at[idx_ref.at[pl.ds(c * GCH, GCH)]],
            sbuf_ref.at[pl.ds(c * GCH, GCH)], sem).wait()


def _gather_body(s_hbm, negv_hbm, posv_hbm, exps_hbm, posp_hbm,
                 idx_ref, sbuf_ref, out_ref, sem):
    core = lax.axis_index("core")
    sub = lax.axis_index("subcore")
    wid = sub * 2 + core

    lane_iota = jax.lax.iota(jnp.int32, LANES)

    pltpu.sync_copy(negv_hbm.at[wid], idx_ref)
    _fetch_s(s_hbm, idx_ref, sbuf_ref, sem)

    expacc = jnp.zeros((LANES,), jnp.float32)
    for gb in range(GPW // LANES):
        offs0 = (gb * LANES + lane_iota) * L

        def jbody(j, acc):
            v = plsc.load_gather(sbuf_ref, [offs0 + j])
            return acc + v

        gsum = lax.fori_loop(0, L, jbody, jnp.zeros((LANES,), jnp.float32))
        nr = gsum * INV
        expacc = expacc + jnp.where(nr > 0.0, jnp.exp(nr), 0.0)

    out_ref[0, :] = expacc
    pltpu.sync_copy(out_ref, exps_hbm.at[pl.ds(wid, 1)])

    pltpu.sync_copy(posv_hbm.at[wid], idx_ref)
    _fetch_s(s_hbm, idx_ref, sbuf_ref, sem)

    def rbody(r, carry):
        return carry + sbuf_ref[pl.ds(r * LANES, LANES)]

    posacc = lax.fori_loop(0, OPW // LANES, rbody,
                           jnp.zeros((LANES,), jnp.float32))
    out_ref[0, :] = posacc
    pltpu.sync_copy(out_ref, posp_hbm.at[pl.ds(wid, 1)])


def _gather_call(s, negv3, posv3):
    mesh = plsc.VectorSubcoreMesh(core_axis_name="core", subcore_axis_name="subcore")
    f = pl.kernel(
        _gather_body,
        compiler_params=pltpu.CompilerParams(
            use_tc_tiling_on_sc=False, needs_layout_passes=False),
        out_type=[
            jax.ShapeDtypeStruct((NW, LANES), jnp.float32),
            jax.ShapeDtypeStruct((NW, LANES), jnp.float32),
        ],
        mesh=mesh,
        scratch_types=[
            pltpu.VMEM((OPW,), jnp.int32),
            pltpu.VMEM((OPW,), jnp.float32),
            pltpu.VMEM((1, LANES), jnp.float32),
            pltpu.SemaphoreType.DMA,
        ],
    )
    return f(s, negv3, posv3)


def kernel(pos_u, pos_v, neg_v, table):
    table_t = table.T
    posu = pos_u.astype(jnp.int32)
    negv3 = neg_v.astype(jnp.int32).reshape(NW, OPW)
    posv3 = pos_v.astype(jnp.int32).reshape(NW, OPW)

    node = _node_call(posu, table_t)
    s = _sweep_call(table_t, node)
    exps, posp = _gather_call(s, negv3, posv3)

    neg_s = jnp.sum(exps)
    pos_loss = jnp.sum(posp) * INV
    return jnp.log(1.0 + neg_s) - pos_loss


# speedup vs baseline: 6.5674x; 2.0270x over previous
import jax
import jax.numpy as jnp
from jax import lax
from jax.experimental import pallas as pl
from jax.experimental.pallas import tpu as pltpu
from jax.experimental.pallas import tpu_sc as plsc

VOCAB = 1000000
DIM = 64
P = 4096
L = 50
LU = 200
NW = 32
GPW = P // NW
OPW = GPW * L
LANES = 16
INV = 1.0 / (float(L) * float(P))
SWEEP_BLK = 16384
NODE_WAY = 8


def _node_body(idx_ref, *refs):
    tbl_refs, out_ref = refs[:NODE_WAY], refs[NODE_WAY]
    i = pl.program_id(0)

    @pl.when(i == 0)
    def _():
        out_ref[...] = jnp.zeros_like(out_ref)

    acc = jnp.zeros((DIM, 1), jnp.float32)
    lanes = jax.lax.broadcasted_iota(jnp.int32, (DIM, 128), 1)
    for k in range(NODE_WAY):
        lane = idx_ref[i * NODE_WAY + k] % 128
        col = jnp.sum(jnp.where(lanes == lane, tbl_refs[k][...], 0.0),
                      axis=1, keepdims=True)
        acc = acc + col
    out_ref[...] += acc

    @pl.when(i == LU // NODE_WAY - 1)
    def _():
        out_ref[...] = out_ref[...] / float(LU)


def _node_call(pos_u, table_t):
    def mk_spec(k):
        return pl.BlockSpec(
            (DIM, 128),
            lambda i, idx_ref, k=k: (0, idx_ref[i * NODE_WAY + k] // 128))

    return pl.pallas_call(
        _node_body,
        grid_spec=pltpu.PrefetchScalarGridSpec(
            num_scalar_prefetch=1,
            grid=(LU // NODE_WAY,),
            in_specs=[mk_spec(k) for k in range(NODE_WAY)],
            out_specs=pl.BlockSpec((DIM, 1), lambda i, idx_ref: (0, 0)),
        ),
        out_shape=jax.ShapeDtypeStruct((DIM, 1), jnp.float32),
    )(pos_u, *([table_t] * NODE_WAY))


def _sweep_body(tbl_ref, node_ref, s_ref):
    s_ref[...] = jnp.sum(tbl_ref[...] * node_ref[...], axis=0)


def _sweep_call(table_t, node):
    grid = (VOCAB + SWEEP_BLK - 1) // SWEEP_BLK
    return pl.pallas_call(
        _sweep_body,
        grid=(grid,),
        in_specs=[
            pl.BlockSpec((DIM, SWEEP_BLK), lambda i: (0, i)),
            pl.BlockSpec((DIM, 1), lambda i: (0, 0)),
        ],
        out_specs=pl.BlockSpec((SWEEP_BLK,), lambda i: (i,)),
        out_shape=jax.ShapeDtypeStruct((VOCAB,), jnp.float32),
    )(table_t, node)


GCH = 128
NCH = OPW // GCH


def _fetch_s(s_hbm, idx_ref, sbuf_ref, sem):
    @pl.loop(0, NCH)
    def _fire(c):
        pltpu.async_copy(
            s_hbm.at[idx_ref.at[pl.ds(c * GCH, GCH)]],
            sbuf_ref.at[pl.ds(c * GCH, GCH)], sem)

    @pl.loop(0, NCH)
    def _drain(c):
        pltpu.make_async_copy(
            s_hbm.at[idx_ref.at[pl.ds(c * GCH, GCH)]],
            sbuf_ref.at[pl.ds(c * GCH, GCH)], sem).wait()


def _gather_body(s_hbm, negv_hbm, posv_hbm, exps_hbm, posp_hbm,
                 idx_ref, sbuf_ref, out_ref, sem):
    core = lax.axis_index("core")
    sub = lax.axis_index("subcore")
    wid = sub * 2 + core

    lane_iota = jax.lax.iota(jnp.int32, LANES)

    pltpu.sync_copy(negv_hbm.at[wid], idx_ref)
    _fetch_s(s_hbm, idx_ref, sbuf_ref, sem)

    expacc = jnp.zeros((LANES,), jnp.float32)
    for gb in range(GPW // LANES):
        offs0 = (gb * LANES + lane_iota) * L

        def jbody(j, acc):
            v = plsc.load_gather(sbuf_ref, [offs0 + j])
            return acc + v

        gsum = lax.fori_loop(0, L, jbody, jnp.zeros((LANES,), jnp.float32))
        nr = gsum * INV
        expacc = expacc + jnp.where(nr > 0.0, jnp.exp(nr), 0.0)

    out_ref[0, :] = expacc
    pltpu.sync_copy(out_ref, exps_hbm.at[pl.ds(wid, 1)])

    pltpu.sync_copy(posv_hbm.at[wid], idx_ref)
    _fetch_s(s_hbm, idx_ref, sbuf_ref, sem)

    def rbody(r, carry):
        return carry + sbuf_ref[pl.ds(r * LANES, LANES)]

    posacc = lax.fori_loop(0, OPW // LANES, rbody,
                           jnp.zeros((LANES,), jnp.float32))
    out_ref[0, :] = posacc
    pltpu.sync_copy(out_ref, posp_hbm.at[pl.ds(wid, 1)])


def _gather_call(s, negv3, posv3):
    mesh = plsc.VectorSubcoreMesh(core_axis_name="core", subcore_axis_name="subcore")
    f = pl.kernel(
        _gather_body,
        compiler_params=pltpu.CompilerParams(
            use_tc_tiling_on_sc=False, needs_layout_passes=False),
        out_type=[
            jax.ShapeDtypeStruct((NW, LANES), jnp.float32),
            jax.ShapeDtypeStruct((NW, LANES), jnp.float32),
        ],
        mesh=mesh,
        scratch_types=[
            pltpu.VMEM((OPW,), jnp.int32),
            pltpu.VMEM((OPW,), jnp.float32),
            pltpu.VMEM((1, LANES), jnp.float32),
            pltpu.SemaphoreType.DMA,
        ],
    )
    return f(s, negv3, posv3)


def kernel(pos_u, pos_v, neg_v, table):
    table_t = table.T
    posu = pos_u.astype(jnp.int32)
    negv3 = neg_v.astype(jnp.int32).reshape(NW, OPW)
    posv3 = pos_v.astype(jnp.int32).reshape(NW, OPW)

    node = _node_call(posu, table_t)
    s = _sweep_call(table_t, node)
    exps, posp = _gather_call(s, negv3, posv3)

    neg_s = jnp.sum(exps)
    pos_loss = jnp.sum(posp) * INV
    return jnp.log(1.0 + neg_s) - pos_loss


# speedup vs baseline: 7.3511x; 1.1193x over previous
import jax
import jax.numpy as jnp
from jax import lax
from jax.experimental import pallas as pl
from jax.experimental.pallas import tpu as pltpu
from jax.experimental.pallas import tpu_sc as plsc

VOCAB = 1000000
DIM = 64
P = 4096
L = 50
LU = 200
NW = 32
GPW = P // NW
OPW = GPW * L
LANES = 16
INV = 1.0 / (float(L) * float(P))
SWEEP_BLK = 32768
NODE_WAY = 8


def _node_body(idx_ref, *refs):
    tbl_refs, out_ref = refs[:NODE_WAY], refs[NODE_WAY]
    i = pl.program_id(0)

    @pl.when(i == 0)
    def _():
        out_ref[...] = jnp.zeros_like(out_ref)

    acc = jnp.zeros((DIM, 1), jnp.float32)
    lanes = jax.lax.broadcasted_iota(jnp.int32, (DIM, 128), 1)
    for k in range(NODE_WAY):
        lane = idx_ref[i * NODE_WAY + k] % 128
        col = jnp.sum(jnp.where(lanes == lane, tbl_refs[k][...], 0.0),
                      axis=1, keepdims=True)
        acc = acc + col
    out_ref[...] += acc

    @pl.when(i == LU // NODE_WAY - 1)
    def _():
        out_ref[...] = out_ref[...] / float(LU)


def _node_call(pos_u, table_t):
    def mk_spec(k):
        return pl.BlockSpec(
            (DIM, 128),
            lambda i, idx_ref, k=k: (0, idx_ref[i * NODE_WAY + k] // 128))

    return pl.pallas_call(
        _node_body,
        grid_spec=pltpu.PrefetchScalarGridSpec(
            num_scalar_prefetch=1,
            grid=(LU // NODE_WAY,),
            in_specs=[mk_spec(k) for k in range(NODE_WAY)],
            out_specs=pl.BlockSpec((DIM, 1), lambda i, idx_ref: (0, 0)),
        ),
        out_shape=jax.ShapeDtypeStruct((DIM, 1), jnp.float32),
    )(pos_u, *([table_t] * NODE_WAY))


def _sweep_body(tbl_ref, node_ref, s_ref):
    s_ref[...] = jnp.sum(tbl_ref[...] * node_ref[...], axis=0)


def _sweep_call(table_t, node):
    grid = (VOCAB + SWEEP_BLK - 1) // SWEEP_BLK
    return pl.pallas_call(
        _sweep_body,
        grid=(grid,),
        in_specs=[
            pl.BlockSpec((DIM, SWEEP_BLK), lambda i: (0, i)),
            pl.BlockSpec((DIM, 1), lambda i: (0, 0)),
        ],
        out_specs=pl.BlockSpec((SWEEP_BLK,), lambda i: (i,)),
        out_shape=jax.ShapeDtypeStruct((VOCAB,), jnp.float32),
    )(table_t, node)


GCH = 128
NCH = OPW // GCH


def _fetch_s(s_hbm, idx_ref, sbuf_ref, sem):
    @pl.loop(0, NCH)
    def _fire(c):
        pltpu.async_copy(
            s_hbm.at[idx_ref.at[pl.ds(c * GCH, GCH)]],
            sbuf_ref.at[pl.ds(c * GCH, GCH)], sem)

    @pl.loop(0, NCH)
    def _drain(c):
        pltpu.make_async_copy(
            s_hbm.at[idx_ref.at[pl.ds(c * GCH, GCH)]],
            sbuf_ref.at[pl.ds(c * GCH, GCH)], sem).wait()


def _gather_body(s_hbm, negv_hbm, posv_hbm, exps_hbm, posp_hbm,
                 idxn_ref, idxp_ref, sbufn_ref, sbufp_ref, out_ref,
                 semn, semp, semi):
    core = lax.axis_index("core")
    sub = lax.axis_index("subcore")
    wid = sub * 2 + core

    lane_iota = jax.lax.iota(jnp.int32, LANES)

    cpn = pltpu.async_copy(negv_hbm.at[wid], idxn_ref, semi)
    cpp = pltpu.async_copy(posv_hbm.at[wid], idxp_ref, semi)
    cpn.wait()

    @pl.loop(0, NCH)
    def _fire_n(c):
        pltpu.async_copy(
            s_hbm.at[idxn_ref.at[pl.ds(c * GCH, GCH)]],
            sbufn_ref.at[pl.ds(c * GCH, GCH)], semn)

    cpp.wait()

    @pl.loop(0, NCH)
    def _fire_p(c):
        pltpu.async_copy(
            s_hbm.at[idxp_ref.at[pl.ds(c * GCH, GCH)]],
            sbufp_ref.at[pl.ds(c * GCH, GCH)], semp)

    @pl.loop(0, NCH)
    def _drain_n(c):
        pltpu.make_async_copy(
            s_hbm.at[idxn_ref.at[pl.ds(c * GCH, GCH)]],
            sbufn_ref.at[pl.ds(c * GCH, GCH)], semn).wait()

    expacc = jnp.zeros((LANES,), jnp.float32)
    for gb in range(GPW // LANES):
        offs0 = (gb * LANES + lane_iota) * L

        def jbody(j, acc):
            v = plsc.load_gather(sbufn_ref, [offs0 + j])
            return acc + v

        gsum = lax.fori_loop(0, L, jbody, jnp.zeros((LANES,), jnp.float32))
        nr = gsum * INV
        expacc = expacc + jnp.where(nr > 0.0, jnp.exp(nr), 0.0)

    out_ref[0, :] = expacc
    pltpu.sync_copy(out_ref, exps_hbm.at[pl.ds(wid, 1)])

    @pl.loop(0, NCH)
    def _drain_p(c):
        pltpu.make_async_copy(
            s_hbm.at[idxp_ref.at[pl.ds(c * GCH, GCH)]],
            sbufp_ref.at[pl.ds(c * GCH, GCH)], semp).wait()

    def rbody(r, carry):
        return carry + sbufp_ref[pl.ds(r * LANES, LANES)]

    posacc = lax.fori_loop(0, OPW // LANES, rbody,
                           jnp.zeros((LANES,), jnp.float32))
    out_ref[0, :] = posacc
    pltpu.sync_copy(out_ref, posp_hbm.at[pl.ds(wid, 1)])


def _gather_call(s, negv3, posv3):
    mesh = plsc.VectorSubcoreMesh(core_axis_name="core", subcore_axis_name="subcore")
    f = pl.kernel(
        _gather_body,
        compiler_params=pltpu.CompilerParams(
            use_tc_tiling_on_sc=False, needs_layout_passes=False),
        out_type=[
            jax.ShapeDtypeStruct((NW, LANES), jnp.float32),
            jax.ShapeDtypeStruct((NW, LANES), jnp.float32),
        ],
        mesh=mesh,
        scratch_types=[
            pltpu.VMEM((OPW,), jnp.int32),
            pltpu.VMEM((OPW,), jnp.int32),
            pltpu.VMEM((OPW,), jnp.float32),
            pltpu.VMEM((OPW,), jnp.float32),
            pltpu.VMEM((1, LANES), jnp.float32),
            pltpu.SemaphoreType.DMA,
            pltpu.SemaphoreType.DMA,
            pltpu.SemaphoreType.DMA,
        ],
    )
    return f(s, negv3, posv3)


def kernel(pos_u, pos_v, neg_v, table):
    table_t = table.T
    posu = pos_u.astype(jnp.int32)
    negv3 = neg_v.astype(jnp.int32).reshape(NW, OPW)
    posv3 = pos_v.astype(jnp.int32).reshape(NW, OPW)

    node = _node_call(posu, table_t)
    s = _sweep_call(table_t, node)
    exps, posp = _gather_call(s, negv3, posv3)

    neg_s = jnp.sum(exps)
    pos_loss = jnp.sum(posp) * INV
    return jnp.log(1.0 + neg_s) - pos_loss


# speedup vs baseline: 7.5541x; 1.0276x over previous
import jax
import jax.numpy as jnp
from jax import lax
from jax.experimental import pallas as pl
from jax.experimental.pallas import tpu as pltpu
from jax.experimental.pallas import tpu_sc as plsc

VOCAB = 1000000
DIM = 64
P = 4096
L = 50
LU = 200
NW = 32
GPW = P // NW
OPW = GPW * L
LANES = 16
INV = 1.0 / (float(L) * float(P))
SWEEP_BLK = 65536
NODE_WAY = 20


def _node_body(idx_ref, *refs):
    tbl_refs, out_ref = refs[:NODE_WAY], refs[NODE_WAY]
    i = pl.program_id(0)

    @pl.when(i == 0)
    def _():
        out_ref[...] = jnp.zeros_like(out_ref)

    acc = jnp.zeros((DIM, 1), jnp.float32)
    lanes = jax.lax.broadcasted_iota(jnp.int32, (DIM, 128), 1)
    for k in range(NODE_WAY):
        lane = idx_ref[i * NODE_WAY + k] % 128
        col = jnp.sum(jnp.where(lanes == lane, tbl_refs[k][...], 0.0),
                      axis=1, keepdims=True)
        acc = acc + col
    out_ref[...] += acc

    @pl.when(i == LU // NODE_WAY - 1)
    def _():
        out_ref[...] = out_ref[...] / float(LU)


def _node_call(pos_u, table_t):
    def mk_spec(k):
        return pl.BlockSpec(
            (DIM, 128),
            lambda i, idx_ref, k=k: (0, idx_ref[i * NODE_WAY + k] // 128))

    return pl.pallas_call(
        _node_body,
        grid_spec=pltpu.PrefetchScalarGridSpec(
            num_scalar_prefetch=1,
            grid=(LU // NODE_WAY,),
            in_specs=[mk_spec(k) for k in range(NODE_WAY)],
            out_specs=pl.BlockSpec((DIM, 1), lambda i, idx_ref: (0, 0)),
        ),
        out_shape=jax.ShapeDtypeStruct((DIM, 1), jnp.float32),
    )(pos_u, *([table_t] * NODE_WAY))


def _sweep_body(tbl_ref, node_ref, s_ref):
    s_ref[...] = jnp.sum(tbl_ref[...] * node_ref[...], axis=0)


def _sweep_call(table_t, node):
    grid = (VOCAB + SWEEP_BLK - 1) // SWEEP_BLK
    return pl.pallas_call(
        _sweep_body,
        grid=(grid,),
        in_specs=[
            pl.BlockSpec((DIM, SWEEP_BLK), lambda i: (0, i)),
            pl.BlockSpec((DIM, 1), lambda i: (0, 0)),
        ],
        out_specs=pl.BlockSpec((SWEEP_BLK,), lambda i: (i,)),
        out_shape=jax.ShapeDtypeStruct((VOCAB,), jnp.float32),
    )(table_t, node)


GCH = 128
NCH = OPW // GCH


def _fetch_s(s_hbm, idx_ref, sbuf_ref, sem):
    @pl.loop(0, NCH)
    def _fire(c):
        pltpu.async_copy(
            s_hbm.at[idx_ref.at[pl.ds(c * GCH, GCH)]],
            sbuf_ref.at[pl.ds(c * GCH, GCH)], sem)

    @pl.loop(0, NCH)
    def _drain(c):
        pltpu.make_async_copy(
            s_hbm.at[idx_ref.at[pl.ds(c * GCH, GCH)]],
            sbuf_ref.at[pl.ds(c * GCH, GCH)], sem).wait()


def _gather_body(s_hbm, negv_hbm, posv_hbm, exps_hbm, posp_hbm,
                 idxn_ref, idxp_ref, sbufn_ref, sbufp_ref, out_ref,
                 semn, semp, semi):
    core = lax.axis_index("core")
    sub = lax.axis_index("subcore")
    wid = sub * 2 + core

    lane_iota = jax.lax.iota(jnp.int32, LANES)

    cpn = pltpu.async_copy(negv_hbm.at[wid], idxn_ref, semi)
    cpp = pltpu.async_copy(posv_hbm.at[wid], idxp_ref, semi)
    cpn.wait()

    @pl.loop(0, NCH)
    def _fire_n(c):
        pltpu.async_copy(
            s_hbm.at[idxn_ref.at[pl.ds(c * GCH, GCH)]],
            sbufn_ref.at[pl.ds(c * GCH, GCH)], semn)

    cpp.wait()

    @pl.loop(0, NCH)
    def _fire_p(c):
        pltpu.async_copy(
            s_hbm.at[idxp_ref.at[pl.ds(c * GCH, GCH)]],
            sbufp_ref.at[pl.ds(c * GCH, GCH)], semp)

    @pl.loop(0, NCH)
    def _drain_n(c):
        pltpu.make_async_copy(
            s_hbm.at[idxn_ref.at[pl.ds(c * GCH, GCH)]],
            sbufn_ref.at[pl.ds(c * GCH, GCH)], semn).wait()

    expacc = jnp.zeros((LANES,), jnp.float32)
    for gb in range(GPW // LANES):
        offs0 = (gb * LANES + lane_iota) * L

        def jbody(j, acc):
            v = plsc.load_gather(sbufn_ref, [offs0 + j])
            return acc + v

        gsum = lax.fori_loop(0, L, jbody, jnp.zeros((LANES,), jnp.float32))
        nr = gsum * INV
        expacc = expacc + jnp.where(nr > 0.0, jnp.exp(nr), 0.0)

    out_ref[0, :] = expacc
    pltpu.sync_copy(out_ref, exps_hbm.at[pl.ds(wid, 1)])

    @pl.loop(0, NCH)
    def _drain_p(c):
        pltpu.make_async_copy(
            s_hbm.at[idxp_ref.at[pl.ds(c * GCH, GCH)]],
            sbufp_ref.at[pl.ds(c * GCH, GCH)], semp).wait()

    def rbody(r, carry):
        return carry + sbufp_ref[pl.ds(r * LANES, LANES)]

    posacc = lax.fori_loop(0, OPW // LANES, rbody,
                           jnp.zeros((LANES,), jnp.float32))
    out_ref[0, :] = posacc
    pltpu.sync_copy(out_ref, posp_hbm.at[pl.ds(wid, 1)])


def _gather_call(s, negv3, posv3):
    mesh = plsc.VectorSubcoreMesh(core_axis_name="core", subcore_axis_name="subcore")
    f = pl.kernel(
        _gather_body,
        compiler_params=pltpu.CompilerParams(
            use_tc_tiling_on_sc=False, needs_layout_passes=False),
        out_type=[
            jax.ShapeDtypeStruct((NW, LANES), jnp.float32),
            jax.ShapeDtypeStruct((NW, LANES), jnp.float32),
        ],
        mesh=mesh,
        scratch_types=[
            pltpu.VMEM((OPW,), jnp.int32),
            pltpu.VMEM((OPW,), jnp.int32),
            pltpu.VMEM((OPW,), jnp.float32),
            pltpu.VMEM((OPW,), jnp.float32),
            pltpu.VMEM((1, LANES), jnp.float32),
            pltpu.SemaphoreType.DMA,
            pltpu.SemaphoreType.DMA,
            pltpu.SemaphoreType.DMA,
        ],
    )
    return f(s, negv3, posv3)


def kernel(pos_u, pos_v, neg_v, table):
    table_t = table.T
    posu = pos_u.astype(jnp.int32)
    negv3 = neg_v.astype(jnp.int32).reshape(NW, OPW)
    posv3 = pos_v.astype(jnp.int32).reshape(NW, OPW)

    node = _node_call(posu, table_t)
    s = _sweep_call(table_t, node)
    exps, posp = _gather_call(s, negv3, posv3)

    neg_s = jnp.sum(exps)
    pos_loss = jnp.sum(posp) * INV
    return jnp.log(1.0 + neg_s) - pos_loss


# speedup vs baseline: 8.0574x; 1.0666x over previous
import jax
import jax.numpy as jnp
from jax import lax
from jax.experimental import pallas as pl
from jax.experimental.pallas import tpu as pltpu
from jax.experimental.pallas import tpu_sc as plsc

VOCAB = 1000000
DIM = 64
P = 4096
L = 50
LU = 200
NW = 32
GPW = P // NW
OPW = GPW * L
LANES = 16
INV = 1.0 / (float(L) * float(P))
SWEEP_BLK = 65536
NODE_WAY = 20


def _node_body(idx_ref, *refs):
    tbl_refs, out_ref = refs[:NODE_WAY], refs[NODE_WAY]
    i = pl.program_id(0)

    @pl.when(i == 0)
    def _():
        out_ref[...] = jnp.zeros_like(out_ref)

    acc = jnp.zeros((DIM, 1), jnp.float32)
    lanes = jax.lax.broadcasted_iota(jnp.int32, (DIM, 128), 1)
    for k in range(NODE_WAY):
        lane = idx_ref[i * NODE_WAY + k] % 128
        col = jnp.sum(jnp.where(lanes == lane, tbl_refs[k][...], 0.0),
                      axis=1, keepdims=True)
        acc = acc + col
    out_ref[...] += acc

    @pl.when(i == LU // NODE_WAY - 1)
    def _():
        out_ref[...] = out_ref[...] / float(LU)


def _node_call(pos_u, table_t):
    def mk_spec(k):
        return pl.BlockSpec(
            (DIM, 128),
            lambda i, idx_ref, k=k: (0, idx_ref[i * NODE_WAY + k] // 128))

    return pl.pallas_call(
        _node_body,
        grid_spec=pltpu.PrefetchScalarGridSpec(
            num_scalar_prefetch=1,
            grid=(LU // NODE_WAY,),
            in_specs=[mk_spec(k) for k in range(NODE_WAY)],
            out_specs=pl.BlockSpec((DIM, 1), lambda i, idx_ref: (0, 0)),
        ),
        out_shape=jax.ShapeDtypeStruct((DIM, 1), jnp.float32),
    )(pos_u, *([table_t] * NODE_WAY))


def _sweep_body(tbl_ref, node_ref, s_ref):
    s_ref[...] = jnp.sum(tbl_ref[...] * node_ref[...], axis=0)


def _sweep_call(table_t, node):
    grid = (VOCAB + SWEEP_BLK - 1) // SWEEP_BLK
    return pl.pallas_call(
        _sweep_body,
        compiler_params=pltpu.CompilerParams(vmem_limit_bytes=56 * 1024 * 1024),
        grid=(grid,),
        in_specs=[
            pl.BlockSpec((DIM, SWEEP_BLK), lambda i: (0, i)),
            pl.BlockSpec((DIM, 1), lambda i: (0, 0)),
        ],
        out_specs=pl.BlockSpec((SWEEP_BLK,), lambda i: (i,)),
        out_shape=jax.ShapeDtypeStruct((VOCAB,), jnp.float32),
    )(table_t, node)


def _gather_body(s_hbm, negv_hbm, posv_hbm, exps_hbm, posp_hbm,
                 idxn_ref, idxp_ref, sbufn_ref, sbufp_ref, out_ref,
                 semn, semp, semi):
    core = lax.axis_index("core")
    sub = lax.axis_index("subcore")
    wid = sub * 2 + core
    col0 = wid * GPW

    cpn = pltpu.async_copy(negv_hbm.at[:, pl.ds(col0, GPW)], idxn_ref, semi)
    cpp = pltpu.async_copy(posv_hbm.at[:, pl.ds(col0, GPW)], idxp_ref, semi)
    cpn.wait()

    @pl.loop(0, L)
    def _fire_n(r):
        pltpu.async_copy(s_hbm.at[idxn_ref.at[r]], sbufn_ref.at[r], semn)

    cpp.wait()

    @pl.loop(0, L)
    def _fire_p(r):
        pltpu.async_copy(s_hbm.at[idxp_ref.at[r]], sbufp_ref.at[r], semp)

    @pl.loop(0, L)
    def _drain_n(r):
        pltpu.make_async_copy(
            s_hbm.at[idxn_ref.at[r]], sbufn_ref.at[r], semn).wait()

    expacc = jnp.zeros((LANES,), jnp.float32)
    for lc in range(GPW // LANES):

        def jbody(r, acc):
            return acc + sbufn_ref[r, pl.ds(lc * LANES, LANES)]

        gsum = lax.fori_loop(0, L, jbody, jnp.zeros((LANES,), jnp.float32))
        nr = gsum * INV
        expacc = expacc + jnp.where(nr > 0.0, jnp.exp(nr), 0.0)

    out_ref[0, :] = expacc
    pltpu.sync_copy(out_ref, exps_hbm.at[pl.ds(wid, 1)])

    @pl.loop(0, L)
    def _drain_p(r):
        pltpu.make_async_copy(
            s_hbm.at[idxp_ref.at[r]], sbufp_ref.at[r], semp).wait()

    def rbody(r, carry):
        acc = carry
        for lc in range(GPW // LANES):
            acc = acc + sbufp_ref[r, pl.ds(lc * LANES, LANES)]
        return acc

    posacc = lax.fori_loop(0, L, rbody, jnp.zeros((LANES,), jnp.float32))
    out_ref[0, :] = posacc
    pltpu.sync_copy(out_ref, posp_hbm.at[pl.ds(wid, 1)])


def _gather_call(s, negv_t, posv_t):
    mesh = plsc.VectorSubcoreMesh(core_axis_name="core", subcore_axis_name="subcore")
    f = pl.kernel(
        _gather_body,
        compiler_params=pltpu.CompilerParams(
            use_tc_tiling_on_sc=True, needs_layout_passes=False),
        out_type=[
            jax.ShapeDtypeStruct((NW, LANES), jnp.float32),
            jax.ShapeDtypeStruct((NW, LANES), jnp.float32),
        ],
        mesh=mesh,
        scratch_types=[
            pltpu.VMEM((L, GPW), jnp.int32),
            pltpu.VMEM((L, GPW), jnp.int32),
            pltpu.VMEM((L, GPW), jnp.float32),
            pltpu.VMEM((L, GPW), jnp.float32),
            pltpu.VMEM((1, LANES), jnp.float32),
            pltpu.SemaphoreType.DMA,
            pltpu.SemaphoreType.DMA,
            pltpu.SemaphoreType.DMA,
        ],
    )
    return f(s, negv_t, posv_t)


def kernel(pos_u, pos_v, neg_v, table):
    table_t = table.T
    posu = pos_u.astype(jnp.int32)
    negv_t = neg_v.astype(jnp.int32).T
    posv_t = pos_v.astype(jnp.int32).T

    node = _node_call(posu, table_t)
    s = _sweep_call(table_t, node)
    exps, posp = _gather_call(s, negv_t, posv_t)

    neg_s = jnp.sum(exps)
    pos_loss = jnp.sum(posp) * INV
    return jnp.log(1.0 + neg_s) - pos_loss
